# baseline (speedup 1.0000x reference)
import numpy as np
import jax
import jax.numpy as jnp
from jax.experimental import pallas as pl
from jax.experimental.pallas import tpu as pltpu

_LIMBS_FROM = np.array(
    [0, 1, 2, 3, 4, 5, 6, 3, 8, 9, 10, 3, 12, 13, 14, 0, 16, 17, 18, 0, 20, 21],
    dtype=np.int32,
)
_LIMBS_TO = np.arange(1, 23, dtype=np.int32)
_NUM_LIMBS = 22


def _selection_matrix() -> np.ndarray:
    sel = np.zeros((23, 128), dtype=np.float32)
    for l in range(_NUM_LIMBS):
        sel[_LIMBS_FROM[l], l] += 1.0
        sel[_LIMBS_TO[l], l] -= 1.0
    return sel


def _loss_kernel(gt_ref, pr_ref, sel_ref, out_ref):
    i = pl.program_id(0)
    d = gt_ref[...] - pr_ref[...]
    sel = sel_ref[...]
    acc = None
    for c in range(3):
        y = jnp.dot(d[:, c, :], sel, preferred_element_type=jnp.float32)
        y = y * y
        acc = y if acc is None else acc + y
    part = jnp.sum(jnp.sqrt(acc)).reshape(1, 1)

    @pl.when(i == 0)
    def _():
        out_ref[...] = jnp.zeros((1, 1), jnp.float32)

    out_ref[...] += part


def kernel(kpts_gt, kpts_pred):
    n, ncoord, nkpt = kpts_gt.shape
    block_b = 1024
    grid = n // block_b
    sel = jnp.asarray(_selection_matrix())
    total = pl.pallas_call(
        _loss_kernel,
        grid=(grid,),
        in_specs=[
            pl.BlockSpec((block_b, ncoord, nkpt), lambda i: (i, 0, 0)),
            pl.BlockSpec((block_b, ncoord, nkpt), lambda i: (i, 0, 0)),
            pl.BlockSpec((nkpt, 128), lambda i: (0, 0)),
        ],
        out_specs=pl.BlockSpec((1, 1), lambda i: (0, 0)),
        out_shape=jax.ShapeDtypeStruct((1, 1), jnp.float32),
    )(kpts_gt, kpts_pred, sel)
    return total[0, 0] / np.float32(n * _NUM_LIMBS)
